# Initial kernel scaffold; baseline (speedup 1.0000x reference)
import jax
import jax.numpy as jnp
from jax.experimental import pallas as pl


def kernel(x, enhanced_weights, enhanced_index, adj, adj_prue, training, enc1_W, enc1_b, enc1_gamma, enc1_beta, enc2_W, enc2_b, enc2_gamma, enc2_beta, gc1_Wq, gc1_bq, gc1_Wk, gc1_bk, gc1_Wv, gc1_bv, gc1_Ws, gc1_bs, ch0_Wq, ch0_bq, ch0_Wk, ch0_bk, ch0_Wv, ch0_bv, ch0_Ws, ch0_bs, gc2_Wq, gc2_bq, gc2_Wk, gc2_bk, gc2_Wv, gc2_bv, gc2_Ws, gc2_bs, dec_W, dec_b, dec_gamma, dec_beta, cluster):
    raise NotImplementedError("write your pallas kernel here")


# speedup vs baseline: 1.1435x; 1.1435x over previous
import jax
import jax.numpy as jnp
from jax.experimental import pallas as pl

N = 4096
AT = 0.5


def _full_block(x, W, b, gamma, beta):
    h = x @ W.T + b
    mean = jnp.mean(h, axis=0)
    var = jnp.var(h, axis=0)
    h = (h - mean) / jnp.sqrt(var + 1e-4) * gamma + beta
    return jax.nn.elu(h)


def _tconv_sparse(x, ei, Wq, bq, Wk, bk, Wv, bv, Ws, bs):
    src, dst = ei[0], ei[1]
    q = x @ Wq.T + bq
    k = x @ Wk.T + bk
    v = x @ Wv.T + bv
    d = q.shape[1]
    a = jnp.sum(q[dst] * k[src], axis=-1) / jnp.sqrt(float(d))
    m = jax.ops.segment_max(a, dst, num_segments=N)
    m = jnp.where(jnp.isfinite(m), m, 0.0)
    ex = jnp.exp(a - m[dst])
    den = jax.ops.segment_sum(ex, dst, num_segments=N)
    an = ex / (den[dst] + 1e-16)
    out = jax.ops.segment_sum(an[:, None] * v[src], dst, num_segments=N)
    return out + x @ Ws.T + bs, an


def _combine(ei, an, h):
    src, dst = ei[0], ei[1]
    c = jax.ops.segment_sum(an[:, None] * h[dst], src, num_segments=N)
    diag = jax.ops.segment_sum(jnp.where(src == dst, an, 0.0), src, num_segments=N)
    return c - diag[:, None] * h


def kernel(x, enhanced_weights, enhanced_index, adj, adj_prue, training,
           enc1_W, enc1_b, enc1_gamma, enc1_beta, enc2_W, enc2_b, enc2_gamma, enc2_beta,
           gc1_Wq, gc1_bq, gc1_Wk, gc1_bk, gc1_Wv, gc1_bv, gc1_Ws, gc1_bs,
           ch0_Wq, ch0_bq, ch0_Wk, ch0_bk, ch0_Wv, ch0_bv, ch0_Ws, ch0_bs,
           gc2_Wq, gc2_bq, gc2_Wk, gc2_bk, gc2_Wv, gc2_bv, gc2_Ws, gc2_bs,
           dec_W, dec_b, dec_gamma, dec_beta, cluster):
    feat_x = _full_block(x, enc1_W, enc1_b, enc1_gamma, enc1_beta)
    feat_x = _full_block(feat_x, enc2_W, enc2_b, enc2_gamma, enc2_beta)

    h1, a1 = _tconv_sparse(feat_x, adj, gc1_Wq, gc1_bq, gc1_Wk, gc1_bk, gc1_Wv, gc1_bv, gc1_Ws, gc1_bs)
    h1p, a1p = _tconv_sparse(feat_x, adj_prue, gc1_Wq, gc1_bq, gc1_Wk, gc1_bk, gc1_Wv, gc1_bv, gc1_Ws, gc1_bs)
    xh = jax.nn.relu((1.0 - AT) * _combine(adj, a1, h1) + AT * _combine(adj_prue, a1p, h1p))

    x1, aa = _tconv_sparse(xh, adj, ch0_Wq, ch0_bq, ch0_Wk, ch0_bk, ch0_Wv, ch0_bv, ch0_Ws, ch0_bs)
    xp, aap = _tconv_sparse(x1, adj_prue, ch0_Wq, ch0_bq, ch0_Wk, ch0_bk, ch0_Wv, ch0_bv, ch0_Ws, ch0_bs)
    xh = jax.nn.relu((1.0 - AT) * _combine(adj, aa, x1) + AT * _combine(adj_prue, aap, xp))

    mu, a2 = _tconv_sparse(xh, adj, gc2_Wq, gc2_bq, gc2_Wk, gc2_bk, gc2_Wv, gc2_bv, gc2_Ws, gc2_bs)
    mup, a2p = _tconv_sparse(xh, adj_prue, gc2_Wq, gc2_bq, gc2_Wk, gc2_bk, gc2_Wv, gc2_bv, gc2_Ws, gc2_bs)
    mu = (1.0 - AT) * _combine(adj, a2, mu) + AT * _combine(adj_prue, a2p, mup)

    z = jnp.concatenate([feat_x, mu], axis=1)
    de_feat = _full_block(z, dec_W, dec_b, dec_gamma, dec_beta)
    qc = 1.0 / (1.0 + jnp.sum((z[:, None, :] - cluster[None, :, :]) ** 2, axis=2))
    qc = qc / jnp.sum(qc, axis=1, keepdims=True)
    return (z, de_feat, qc, feat_x, z)


# speedup vs baseline: 6.6653x; 5.8287x over previous
import functools

import jax
import jax.numpy as jnp
from jax import lax
from jax.experimental import pallas as pl
from jax.experimental.pallas import tpu as pltpu
from jax.experimental.pallas import tpu_sc as plsc

N = 4096
E = 65536
AT = 0.5
TILES = 16
E_T = E // TILES
EB = 128
NB = E_T // EB
GP = EB // 16
NS = N // TILES
RW = 128
AW = 32

_SC_PARAMS = pltpu.CompilerParams(needs_layout_passes=False)


def _matmul_t(x, w):
    return lax.dot_general(x, w, (((1,), (1,)), ((), ())),
                           preferred_element_type=jnp.float32)


def _norm_elu(h, gamma, beta):
    m = jnp.mean(h, axis=0)
    v = jnp.mean((h - m) ** 2, axis=0)
    h = (h - m) / jnp.sqrt(v + 1e-4) * gamma + beta
    return jnp.where(h > 0, h, jnp.exp(h) - 1.0)


def _tc_encoder(x, w1, b1, g1, be1, w2, b2, g2, be2):
    def body(x_r, w1_r, b1_r, g1_r, be1_r, w2_r, b2_r, g2_r, be2_r, o_r):
        h = _matmul_t(x_r[...], w1_r[...]) + b1_r[...]
        h = _norm_elu(h, g1_r[...], be1_r[...])
        h = _matmul_t(h, w2_r[...]) + b2_r[...]
        o_r[...] = _norm_elu(h, g2_r[...], be2_r[...])

    return pl.pallas_call(
        body,
        out_shape=jax.ShapeDtypeStruct((N, w2.shape[0]), jnp.float32),
    )(x, w1, b1, g1, be1, w2, b2, g2, be2)


def _tc_proj(x, wq, bq, wk, bk, wv, bv, ws, bs):
    dh = wq.shape[0]
    scale = 1.0 / float(dh) ** 0.5

    def body(x_r, wq_r, bq_r, wk_r, bk_r, wv_r, bv_r, ws_r, bs_r,
             q_r, kv_r, s_r):
        xv = x_r[...]
        q = (_matmul_t(xv, wq_r[...]) + bq_r[...]) * scale
        k = _matmul_t(xv, wk_r[...]) + bk_r[...]
        v = _matmul_t(xv, wv_r[...]) + bv_r[...]
        q_r[...] = jnp.concatenate(
            [q, jnp.zeros((N, RW - dh), jnp.float32)], axis=1)
        if 2 * dh == RW:
            kv_r[...] = jnp.concatenate([k, v], axis=1)
        else:
            kv_r[...] = jnp.concatenate(
                [k, v, jnp.zeros((N, RW - 2 * dh), jnp.float32)], axis=1)
        s_r[...] = _matmul_t(xv, ws_r[...]) + bs_r[...]

    return pl.pallas_call(body, out_shape=[
        jax.ShapeDtypeStruct((N, RW), jnp.float32),
        jax.ShapeDtypeStruct((N, RW), jnp.float32),
        jax.ShapeDtypeStruct((N, dh), jnp.float32),
    ])(x, wq, bq, wk, bk, wv, bv, ws, bs)


def _tc_skip(attn, s, dh):
    def body(a_r, s_r, o_r):
        h = a_r[:, dh:2 * dh] + s_r[...]
        o_r[...] = jnp.concatenate(
            [h, jnp.zeros((N, RW - dh), jnp.float32)], axis=1)

    return pl.pallas_call(
        body, out_shape=jax.ShapeDtypeStruct((N, RW), jnp.float32))(attn, s)


def _tc_merge(c, diag, ha, hp, dh, relu):
    def body(c_r, d_r, ha_r, hp_r, o_r):
        d0 = d_r[0, :].reshape(N, 1)
        d1 = d_r[1, :].reshape(N, 1)
        out = ((1.0 - AT) * (c_r[0, :, 0:dh] - d0 * ha_r[:, 0:dh])
               + AT * (c_r[1, :, 0:dh] - d1 * hp_r[:, 0:dh]))
        if relu:
            out = jnp.maximum(out, 0.0)
        o_r[...] = out

    return pl.pallas_call(
        body, out_shape=jax.ShapeDtypeStruct((N, dh), jnp.float32))(
            c, diag, ha, hp)


def _tc_final(feat_x, mu, dw, db, dg, dbe, cluster):
    nclust = cluster.shape[0]

    def body(f_r, mu_r, dw_r, db_r, dg_r, dbe_r, cl_r, z_r, de_r, q_r):
        z = jnp.concatenate([f_r[...], mu_r[...]], axis=1)
        z_r[...] = z
        h = _matmul_t(z, dw_r[...]) + db_r[...]
        de_r[...] = _norm_elu(h, dg_r[...], dbe_r[...])
        cl = cl_r[...]
        cols = []
        for j in range(nclust):
            dif = z - cl[j, :].reshape(1, -1)
            cols.append(1.0 / (1.0 + jnp.sum(dif * dif, axis=1, keepdims=True)))
        q = jnp.concatenate(cols, axis=1)
        q_r[...] = q / jnp.sum(q, axis=1, keepdims=True)

    d = feat_x.shape[1] + mu.shape[1]
    return pl.pallas_call(
        body,
        out_shape=[
            jax.ShapeDtypeStruct((N, d), jnp.float32),
            jax.ShapeDtypeStruct((N, dw.shape[0]), jnp.float32),
            jax.ShapeDtypeStruct((N, nclust), jnp.float32),
        ],
    )(feat_x, mu, dw, db, dg, dbe, cluster)


def _splat(val):
    return jnp.full((16,), val)


@functools.lru_cache(maxsize=None)
def _make_sc_attn(dh):
    mesh = plsc.VectorSubcoreMesh(core_axis_name="c", subcore_axis_name="s")

    def body(qs, kv, ei, an_o, attn_o, fin_o,
             dst_stk, dst_raw, src_stk, a_ch, banks,
             qrows, kvrows, tbuf, abuf, mfin, zbuf,
             accum, sem1, sem2):
        core = lax.axis_index("c")
        sid = lax.axis_index("s")
        lane = jnp.arange(16, dtype=jnp.int32)
        blane = jnp.bitwise_and(lane, 3)
        bms = [jnp.logical_and(lane >= _splat(4 * p), lane < _splat(4 * p + 4))
               for p in range(4)]
        coff = core * N
        base = sid * NS

        pltpu.sync_copy(ei.at[core, 1, sid], dst_raw)
        pltpu.sync_copy(ei.at[core, 0, sid], src_stk)

        def p0(b, c_):
            for j in range(GP):
                dv = dst_raw[b, pl.ds(j * 16, 16)]
                dst_stk[b, pl.ds(j * 16, 16)] = dv + _splat(coff)
                sv = src_stk[b, pl.ds(j * 16, 16)]
                src_stk[b, pl.ds(j * 16, 16)] = sv + _splat(coff)
            return c_
        lax.fori_loop(0, NB, p0, 0)

        for r in range(16):
            for t in range(RW // 16):
                zbuf[r, pl.ds(t * 16, 16)] = jnp.zeros((16,), jnp.float32)

        def p1(b, c_):
            cpq = pltpu.async_copy(qs.at[dst_stk.at[b]], qrows, sem1)
            cpk = pltpu.async_copy(kv.at[src_stk.at[b]], kvrows, sem2)
            cpq.wait()
            cpk.wait()
            for g in range(GP):
                ev = lane + _splat(g * 16)
                acc = (plsc.load_gather(qrows, [ev, _splat(0)])
                       * plsc.load_gather(kvrows, [ev, _splat(0)]))
                for d in range(1, dh):
                    acc = acc + (plsc.load_gather(qrows, [ev, _splat(d)])
                                 * plsc.load_gather(kvrows, [ev, _splat(d)]))
                a_ch[pl.ds(b * EB + g * 16, 16)] = acc
            return c_
        lax.fori_loop(0, NB, p1, 0)

        def initm(i, c_):
            for r in range(4):
                banks[r, pl.ds(i * 16, 16)] = _splat(jnp.float32(-jnp.inf))
            return c_
        lax.fori_loop(0, N // 16, initm, 0)

        def smax(b, c_):
            for j in range(GP):
                dv = dst_raw[b, pl.ds(j * 16, 16)]
                av = a_ch[pl.ds(b * EB + j * 16, 16)]
                for p in range(4):
                    cur = plsc.load_gather(banks, [blane, dv])
                    plsc.store_scatter(banks, [blane, dv],
                                       jnp.maximum(cur, av), mask=bms[p])
            return c_
        lax.fori_loop(0, NB, smax, 0)

        def mred(ci, c_):
            for u in range(NS // 16):
                col = ci * NS + u * 16
                acc = banks[0, pl.ds(col, 16)]
                for r in range(1, 4):
                    acc = jnp.maximum(acc, banks[r, pl.ds(col, 16)])
                abuf[pl.ds(u * 16, 16)] = acc
            pltpu.sync_copy(abuf, an_o.at[core, sid, pl.ds(ci * NS, NS)])
            return c_
        lax.fori_loop(0, TILES, mred, 0)

        plsc.subcore_barrier()
        pltpu.sync_copy(an_o.at[core, 0, pl.ds(base, NS)], abuf)
        for s in range(1, TILES):
            pltpu.sync_copy(an_o.at[core, s, pl.ds(base, NS)], tbuf)
            for u in range(NS // 16):
                abuf[pl.ds(u * 16, 16)] = jnp.maximum(
                    abuf[pl.ds(u * 16, 16)], tbuf[pl.ds(u * 16, 16)])
        pltpu.sync_copy(abuf, fin_o.at[core, pl.ds(base, NS)])
        plsc.subcore_barrier()
        pltpu.sync_copy(fin_o.at[core], mfin)

        def initz(i, c_):
            for r in range(4):
                banks[r, pl.ds(i * 16, 16)] = _splat(jnp.float32(0.0))
            return c_
        lax.fori_loop(0, N // 16, initz, 0)

        def pex(b, c_):
            for j in range(GP):
                dv = dst_raw[b, pl.ds(j * 16, 16)]
                av = a_ch[pl.ds(b * EB + j * 16, 16)]
                mv = plsc.load_gather(mfin, [dv])
                ev = jnp.exp(av - mv)
                a_ch[pl.ds(b * EB + j * 16, 16)] = ev
                for p in range(4):
                    cur = plsc.load_gather(banks, [blane, dv])
                    plsc.store_scatter(banks, [blane, dv], cur + ev,
                                       mask=bms[p])
            return c_
        lax.fori_loop(0, NB, pex, 0)

        def dred(ci, c_):
            for u in range(NS // 16):
                col = ci * NS + u * 16
                acc = banks[0, pl.ds(col, 16)]
                for r in range(1, 4):
                    acc = acc + banks[r, pl.ds(col, 16)]
                abuf[pl.ds(u * 16, 16)] = acc
            pltpu.sync_copy(abuf, an_o.at[core, sid, pl.ds(ci * NS, NS)])
            return c_
        lax.fori_loop(0, TILES, dred, 0)

        plsc.subcore_barrier()
        pltpu.sync_copy(an_o.at[core, 0, pl.ds(base, NS)], abuf)
        for s in range(1, TILES):
            pltpu.sync_copy(an_o.at[core, s, pl.ds(base, NS)], tbuf)
            for u in range(NS // 16):
                abuf[pl.ds(u * 16, 16)] = abuf[pl.ds(u * 16, 16)] + tbuf[pl.ds(u * 16, 16)]
        pltpu.sync_copy(abuf, fin_o.at[core, pl.ds(base, NS)])
        plsc.subcore_barrier()
        pltpu.sync_copy(fin_o.at[core], mfin)

        def pan(b, c_):
            for j in range(GP):
                dv = dst_raw[b, pl.ds(j * 16, 16)]
                ev = a_ch[pl.ds(b * EB + j * 16, 16)]
                denv = plsc.load_gather(mfin, [dv])
                a_ch[pl.ds(b * EB + j * 16, 16)] = ev / (denv + 1e-16)
            return c_
        lax.fori_loop(0, NB, pan, 0)
        pltpu.sync_copy(a_ch, an_o.at[core, sid])

        for j in range(NS // 16):
            pltpu.sync_copy(zbuf, accum.at[pl.ds(base + j * 16, 16)])
        plsc.subcore_barrier()

        def pw(b, c_):
            cpv = pltpu.async_copy(kv.at[src_stk.at[b]], kvrows, sem1)
            cpv.wait()
            for e in range(EB):
                wv = plsc.load_gather(a_ch, [_splat(b * EB + e)])
                for t in range(dh // 16):
                    kvrows[e, pl.ds(dh + t * 16, 16)] = (
                        kvrows[e, pl.ds(dh + t * 16, 16)] * wv)
            pltpu.sync_copy(kvrows, accum.at[dst_raw.at[b]], add=True)
            return c_
        lax.fori_loop(0, NB, pw, 0)
        plsc.subcore_barrier()
        pltpu.sync_copy(accum.at[pl.ds(base, NS)],
                        attn_o.at[core, pl.ds(base, NS)])

    kern = pl.kernel(
        body,
        out_type=[
            jax.ShapeDtypeStruct((2, TILES, E_T), jnp.float32),
            jax.ShapeDtypeStruct((2, N, RW), jnp.float32),
            jax.ShapeDtypeStruct((2, N), jnp.float32),
        ],
        mesh=mesh,
        compiler_params=_SC_PARAMS,
        scratch_types=[
            pltpu.VMEM((NB, EB), jnp.int32),
            pltpu.VMEM((NB, EB), jnp.int32),
            pltpu.VMEM((NB, EB), jnp.int32),
            pltpu.VMEM((E_T,), jnp.float32),
            pltpu.VMEM((4, N), jnp.float32),
            pltpu.VMEM((EB, RW), jnp.float32),
            pltpu.VMEM((EB, RW), jnp.float32),
            pltpu.VMEM((NS,), jnp.float32),
            pltpu.VMEM((NS,), jnp.float32),
            pltpu.VMEM((N,), jnp.float32),
            pltpu.VMEM((16, RW), jnp.float32),
            pltpu.VMEM_SHARED((N, RW), jnp.float32),
            pltpu.SemaphoreType.DMA,
            pltpu.SemaphoreType.DMA,
        ],
    )
    return kern


@functools.lru_cache(maxsize=None)
def _make_sc_combine(dh):
    mesh = plsc.VectorSubcoreMesh(core_axis_name="c", subcore_axis_name="s")

    def body(hs, ei, an_i, c_o, diag_o, stg_o,
             dst_stk, src_raw, an_ch, banks, hrows, tbuf, abuf, zbuf,
             accum, sem1):
        core = lax.axis_index("c")
        sid = lax.axis_index("s")
        lane = jnp.arange(16, dtype=jnp.int32)
        blane = jnp.bitwise_and(lane, 3)
        bms = [jnp.logical_and(lane >= _splat(4 * p), lane < _splat(4 * p + 4))
               for p in range(4)]
        coff = core * N
        base = sid * NS

        pltpu.sync_copy(ei.at[core, 1, sid], dst_stk)
        pltpu.sync_copy(ei.at[core, 0, sid], src_raw)
        pltpu.sync_copy(an_i.at[core, sid], an_ch)

        def p0(b, c_):
            for j in range(GP):
                dv = dst_stk[b, pl.ds(j * 16, 16)]
                dst_stk[b, pl.ds(j * 16, 16)] = dv + _splat(coff)
            return c_
        lax.fori_loop(0, NB, p0, 0)

        def initz(i, c_):
            for r in range(4):
                banks[r, pl.ds(i * 16, 16)] = _splat(jnp.float32(0.0))
            return c_
        lax.fori_loop(0, N // 16, initz, 0)

        for r in range(16):
            for t in range(RW // 16):
                zbuf[r, pl.ds(t * 16, 16)] = jnp.zeros((16,), jnp.float32)
        for j in range(NS // 16):
            pltpu.sync_copy(zbuf, accum.at[pl.ds(base + j * 16, 16)])
        plsc.subcore_barrier()

        def p1(b, c_):
            cph = pltpu.async_copy(hs.at[dst_stk.at[b]], hrows, sem1)
            cph.wait()
            for j in range(GP):
                sv = src_raw[b, pl.ds(j * 16, 16)]
                dvr = dst_stk[b, pl.ds(j * 16, 16)] - _splat(coff)
                av = an_ch[pl.ds(b * EB + j * 16, 16)]
                contrib = jnp.where(sv == dvr, av, jnp.float32(0.0))
                for p in range(4):
                    cur = plsc.load_gather(banks, [blane, sv])
                    plsc.store_scatter(banks, [blane, sv],
                                       cur + contrib, mask=bms[p])
            for e in range(EB):
                wv = plsc.load_gather(an_ch, [_splat(b * EB + e)])
                for t in range(dh // 16):
                    hrows[e, pl.ds(t * 16, 16)] = hrows[e, pl.ds(t * 16, 16)] * wv
            pltpu.sync_copy(hrows, accum.at[src_raw.at[b]], add=True)
            return c_
        lax.fori_loop(0, NB, p1, 0)
        plsc.subcore_barrier()
        pltpu.sync_copy(accum.at[pl.ds(base, NS)],
                        c_o.at[core, pl.ds(base, NS)])

        def dred(ci, c_):
            for u in range(NS // 16):
                col = ci * NS + u * 16
                acc = banks[0, pl.ds(col, 16)]
                for r in range(1, 4):
                    acc = acc + banks[r, pl.ds(col, 16)]
                abuf[pl.ds(u * 16, 16)] = acc
            pltpu.sync_copy(abuf, stg_o.at[core, sid, pl.ds(ci * NS, NS)])
            return c_
        lax.fori_loop(0, TILES, dred, 0)

        plsc.subcore_barrier()
        pltpu.sync_copy(stg_o.at[core, 0, pl.ds(base, NS)], abuf)
        for s in range(1, TILES):
            pltpu.sync_copy(stg_o.at[core, s, pl.ds(base, NS)], tbuf)
            for u in range(NS // 16):
                abuf[pl.ds(u * 16, 16)] = abuf[pl.ds(u * 16, 16)] + tbuf[pl.ds(u * 16, 16)]
        pltpu.sync_copy(abuf, diag_o.at[core, pl.ds(base, NS)])

    kern = pl.kernel(
        body,
        out_type=[
            jax.ShapeDtypeStruct((2, N, RW), jnp.float32),
            jax.ShapeDtypeStruct((2, N), jnp.float32),
            jax.ShapeDtypeStruct((2, TILES, N), jnp.float32),
        ],
        mesh=mesh,
        compiler_params=_SC_PARAMS,
        scratch_types=[
            pltpu.VMEM((NB, EB), jnp.int32),
            pltpu.VMEM((NB, EB), jnp.int32),
            pltpu.VMEM((E_T,), jnp.float32),
            pltpu.VMEM((4, N), jnp.float32),
            pltpu.VMEM((EB, RW), jnp.float32),
            pltpu.VMEM((NS,), jnp.float32),
            pltpu.VMEM((NS,), jnp.float32),
            pltpu.VMEM((16, RW), jnp.float32),
            pltpu.VMEM_SHARED((N, RW), jnp.float32),
            pltpu.SemaphoreType.DMA,
        ],
    )
    return kern


def _pack_ei(a, b):
    return jnp.stack([a, b], axis=0).reshape(2, 2, TILES, NB, EB)


def _stack2(a, b=None):
    if b is None:
        b = a
    return jnp.concatenate([a, b], axis=0)


def kernel(x, enhanced_weights, enhanced_index, adj, adj_prue, training,
           enc1_W, enc1_b, enc1_gamma, enc1_beta, enc2_W, enc2_b, enc2_gamma, enc2_beta,
           gc1_Wq, gc1_bq, gc1_Wk, gc1_bk, gc1_Wv, gc1_bv, gc1_Ws, gc1_bs,
           ch0_Wq, ch0_bq, ch0_Wk, ch0_bk, ch0_Wv, ch0_bv, ch0_Ws, ch0_bs,
           gc2_Wq, gc2_bq, gc2_Wk, gc2_bk, gc2_Wv, gc2_bv, gc2_Ws, gc2_bs,
           dec_W, dec_b, dec_gamma, dec_beta, cluster):
    adj = adj.astype(jnp.int32)
    adj_prue = adj_prue.astype(jnp.int32)
    ei_pair = _pack_ei(adj, adj_prue)
    ei_a = _pack_ei(adj, adj)
    ei_p = _pack_ei(adj_prue, adj_prue)

    feat_x = _tc_encoder(x, enc1_W, enc1_b, enc1_gamma, enc1_beta,
                         enc2_W, enc2_b, enc2_gamma, enc2_beta)

    q, kv, s = _tc_proj(feat_x, gc1_Wq, gc1_bq, gc1_Wk, gc1_bk,
                        gc1_Wv, gc1_bv, gc1_Ws, gc1_bs)
    an1, attn1, _ = _make_sc_attn(64)(_stack2(q), _stack2(kv), ei_pair)
    h1 = _tc_skip(attn1[0], s, 64)
    h1p = _tc_skip(attn1[1], s, 64)
    c1, d1, _ = _make_sc_combine(64)(_stack2(h1, h1p), ei_pair, an1)
    xh = _tc_merge(c1, d1, h1, h1p, 64, relu=True)

    qa, kva, sa = _tc_proj(xh, ch0_Wq, ch0_bq, ch0_Wk, ch0_bk,
                           ch0_Wv, ch0_bv, ch0_Ws, ch0_bs)
    anA, attnA, _ = _make_sc_attn(64)(_stack2(qa), _stack2(kva), ei_a)
    x1 = _tc_skip(attnA[0], sa, 64)
    qb, kvb, sb = _tc_proj(x1[:, 0:64], ch0_Wq, ch0_bq, ch0_Wk, ch0_bk,
                           ch0_Wv, ch0_bv, ch0_Ws, ch0_bs)
    anB, attnB, _ = _make_sc_attn(64)(_stack2(qb), _stack2(kvb), ei_p)
    xp = _tc_skip(attnB[0], sb, 64)
    an2 = jnp.stack([anA[0], anB[0]], axis=0)
    c2, d2, _ = _make_sc_combine(64)(_stack2(x1, xp), ei_pair, an2)
    xh = _tc_merge(c2, d2, x1, xp, 64, relu=True)

    q3, kv3, s3 = _tc_proj(xh, gc2_Wq, gc2_bq, gc2_Wk, gc2_bk,
                           gc2_Wv, gc2_bv, gc2_Ws, gc2_bs)
    an3, attn3, _ = _make_sc_attn(32)(_stack2(q3), _stack2(kv3), ei_pair)
    mu1 = _tc_skip(attn3[0], s3, 32)
    mup = _tc_skip(attn3[1], s3, 32)
    c3, d3, _ = _make_sc_combine(32)(_stack2(mu1, mup), ei_pair, an3)
    mu = _tc_merge(c3, d3, mu1, mup, 32, relu=False)

    z, de_feat, qc = _tc_final(feat_x, mu, dec_W, dec_b, dec_gamma, dec_beta,
                               cluster)
    return (z, de_feat, qc, feat_x, z)


# speedup vs baseline: 7.0490x; 1.0576x over previous
import functools

import jax
import jax.numpy as jnp
from jax import lax
from jax.experimental import pallas as pl
from jax.experimental.pallas import tpu as pltpu
from jax.experimental.pallas import tpu_sc as plsc

N = 4096
E = 65536
AT = 0.5
TILES = 16
E_T = E // TILES
EB = 128
NB = E_T // EB
GP = EB // 16
NS = N // TILES
RW = 128
AW = 32

_SC_PARAMS = pltpu.CompilerParams(needs_layout_passes=False)


def _matmul_t(x, w):
    return lax.dot_general(x, w, (((1,), (1,)), ((), ())),
                           preferred_element_type=jnp.float32)


def _norm_elu(h, gamma, beta):
    m = jnp.mean(h, axis=0)
    v = jnp.mean((h - m) ** 2, axis=0)
    h = (h - m) / jnp.sqrt(v + 1e-4) * gamma + beta
    return jnp.where(h > 0, h, jnp.exp(h) - 1.0)


def _tc_encoder(x, w1, b1, g1, be1, w2, b2, g2, be2):
    def body(x_r, w1_r, b1_r, g1_r, be1_r, w2_r, b2_r, g2_r, be2_r, o_r):
        h = _matmul_t(x_r[...], w1_r[...]) + b1_r[...]
        h = _norm_elu(h, g1_r[...], be1_r[...])
        h = _matmul_t(h, w2_r[...]) + b2_r[...]
        o_r[...] = _norm_elu(h, g2_r[...], be2_r[...])

    return pl.pallas_call(
        body,
        out_shape=jax.ShapeDtypeStruct((N, w2.shape[0]), jnp.float32),
    )(x, w1, b1, g1, be1, w2, b2, g2, be2)


def _tc_proj(x, wq, bq, wk, bk, wv, bv, ws, bs):
    dh = wq.shape[0]
    scale = 1.0 / float(dh) ** 0.5

    def body(x_r, wq_r, bq_r, wk_r, bk_r, wv_r, bv_r, ws_r, bs_r,
             q_r, kv_r, s_r):
        xv = x_r[...]
        q = (_matmul_t(xv, wq_r[...]) + bq_r[...]) * scale
        k = _matmul_t(xv, wk_r[...]) + bk_r[...]
        v = _matmul_t(xv, wv_r[...]) + bv_r[...]
        q_r[...] = jnp.concatenate(
            [q, jnp.zeros((N, RW - dh), jnp.float32)], axis=1)
        if 2 * dh == RW:
            kv_r[...] = jnp.concatenate([k, v], axis=1)
        else:
            kv_r[...] = jnp.concatenate(
                [k, v, jnp.zeros((N, RW - 2 * dh), jnp.float32)], axis=1)
        s_r[...] = _matmul_t(xv, ws_r[...]) + bs_r[...]

    return pl.pallas_call(body, out_shape=[
        jax.ShapeDtypeStruct((N, RW), jnp.float32),
        jax.ShapeDtypeStruct((N, RW), jnp.float32),
        jax.ShapeDtypeStruct((N, dh), jnp.float32),
    ])(x, wq, bq, wk, bk, wv, bv, ws, bs)


def _tc_skip(attn, s, dh):
    def body(a_r, s_r, o_r):
        h = a_r[:, dh:2 * dh] + s_r[...]
        o_r[...] = jnp.concatenate(
            [h, jnp.zeros((N, RW - dh), jnp.float32)], axis=1)

    return pl.pallas_call(
        body, out_shape=jax.ShapeDtypeStruct((N, RW), jnp.float32))(attn, s)


def _tc_merge(c, diag, ha, hp, dh, relu):
    def body(c_r, d_r, ha_r, hp_r, o_r):
        d0 = d_r[0, :].reshape(N, 1)
        d1 = d_r[1, :].reshape(N, 1)
        out = ((1.0 - AT) * (c_r[0, :, 0:dh] - d0 * ha_r[:, 0:dh])
               + AT * (c_r[1, :, 0:dh] - d1 * hp_r[:, 0:dh]))
        if relu:
            out = jnp.maximum(out, 0.0)
        o_r[...] = out

    return pl.pallas_call(
        body, out_shape=jax.ShapeDtypeStruct((N, dh), jnp.float32))(
            c, diag, ha, hp)


def _tc_final(feat_x, mu, dw, db, dg, dbe, cluster):
    nclust = cluster.shape[0]

    def body(f_r, mu_r, dw_r, db_r, dg_r, dbe_r, cl_r, z_r, de_r, q_r):
        z = jnp.concatenate([f_r[...], mu_r[...]], axis=1)
        z_r[...] = z
        h = _matmul_t(z, dw_r[...]) + db_r[...]
        de_r[...] = _norm_elu(h, dg_r[...], dbe_r[...])
        cl = cl_r[...]
        cols = []
        for j in range(nclust):
            dif = z - cl[j, :].reshape(1, -1)
            cols.append(1.0 / (1.0 + jnp.sum(dif * dif, axis=1, keepdims=True)))
        q = jnp.concatenate(cols, axis=1)
        q_r[...] = q / jnp.sum(q, axis=1, keepdims=True)

    d = feat_x.shape[1] + mu.shape[1]
    return pl.pallas_call(
        body,
        out_shape=[
            jax.ShapeDtypeStruct((N, d), jnp.float32),
            jax.ShapeDtypeStruct((N, dw.shape[0]), jnp.float32),
            jax.ShapeDtypeStruct((N, nclust), jnp.float32),
        ],
    )(feat_x, mu, dw, db, dg, dbe, cluster)


def _splat(val):
    return jnp.full((16,), val)


@functools.lru_cache(maxsize=None)
def _make_sc_attn(dh):
    mesh = plsc.VectorSubcoreMesh(core_axis_name="c", subcore_axis_name="s")

    def body(qs, kv, ei, an_o, attn_o, fin_o,
             dst_stk, dst_raw, src_stk, a_ch, banks,
             qrows, qrows2, kvrows2a, kvrows2b, kvrows, stagebuf, abuf,
             mfin, zbuf, accum, sem1, sem2, sem3, sem4):
        core = lax.axis_index("c")
        sid = lax.axis_index("s")
        lane = jnp.arange(16, dtype=jnp.int32)
        blane = jnp.bitwise_and(lane, 3)
        bms = [jnp.logical_and(lane >= _splat(4 * p), lane < _splat(4 * p + 4))
               for p in range(4)]
        coff = core * N
        base = sid * NS

        pltpu.sync_copy(ei.at[core, 1, sid], dst_raw)
        pltpu.sync_copy(ei.at[core, 0, sid], src_stk)

        def p0(b, c_):
            for j in range(GP):
                dv = dst_raw[b, pl.ds(j * 16, 16)]
                dst_stk[b, pl.ds(j * 16, 16)] = dv + _splat(coff)
                sv = src_stk[b, pl.ds(j * 16, 16)]
                src_stk[b, pl.ds(j * 16, 16)] = sv + _splat(coff)
            return c_
        lax.fori_loop(0, NB, p0, 0)

        for r in range(16):
            for t in range(RW // 16):
                zbuf[r, pl.ds(t * 16, 16)] = jnp.zeros((16,), jnp.float32)

        def p1(b, c_):
            qb = (qrows, qrows2)
            kb = (kvrows2a, kvrows2b)
            sq = (sem1, sem3)
            sk = (sem2, sem4)
            cps = [None, None]
            cps[0] = (pltpu.async_copy(
                          qs.at[dst_stk.at[b, pl.ds(0, 32)]], qb[0], sq[0]),
                      pltpu.async_copy(
                          kv.at[src_stk.at[b, pl.ds(0, 32)]], kb[0], sk[0]))
            for o in range(4):
                cur = o % 2
                if o < 3:
                    nxt = 1 - cur
                    cps[nxt] = (
                        pltpu.async_copy(
                            qs.at[dst_stk.at[b, pl.ds((o + 1) * 32, 32)]],
                            qb[nxt], sq[nxt]),
                        pltpu.async_copy(
                            kv.at[src_stk.at[b, pl.ds((o + 1) * 32, 32)]],
                            kb[nxt], sk[nxt]))
                cps[cur][0].wait()
                cps[cur][1].wait()
                for g in range(2):
                    ev = lane + _splat(g * 16)
                    acc = (plsc.load_gather(qb[cur], [ev, _splat(0)])
                           * plsc.load_gather(kb[cur], [ev, _splat(0)]))
                    for d in range(1, dh):
                        acc = acc + (plsc.load_gather(qb[cur], [ev, _splat(d)])
                                     * plsc.load_gather(kb[cur], [ev, _splat(d)]))
                    a_ch[pl.ds(b * EB + o * 32 + g * 16, 16)] = acc
            return c_
        lax.fori_loop(0, NB, p1, 0)

        def initm(i, c_):
            for r in range(4):
                banks[r, pl.ds(i * 16, 16)] = _splat(jnp.float32(-jnp.inf))
            return c_
        lax.fori_loop(0, N // 16, initm, 0)

        def smax(b, c_):
            for j in range(GP):
                dv = dst_raw[b, pl.ds(j * 16, 16)]
                av = a_ch[pl.ds(b * EB + j * 16, 16)]
                for p in range(4):
                    cur = plsc.load_gather(banks, [blane, dv])
                    plsc.store_scatter(banks, [blane, dv],
                                       jnp.maximum(cur, av), mask=bms[p])
            return c_
        lax.fori_loop(0, NB, smax, 0)

        def mred(ci, c_):
            for u in range(NS // 16):
                col = ci * NS + u * 16
                acc = banks[0, pl.ds(col, 16)]
                for r in range(1, 4):
                    acc = jnp.maximum(acc, banks[r, pl.ds(col, 16)])
                abuf[pl.ds(u * 16, 16)] = acc
            pltpu.sync_copy(abuf, an_o.at[core, sid, pl.ds(ci * NS, NS)])
            return c_
        lax.fori_loop(0, TILES, mred, 0)

        plsc.subcore_barrier()
        pltpu.sync_copy(an_o.at[core, :, pl.ds(base, NS)], stagebuf)
        for u in range(NS // 16):
            acc = stagebuf[0, pl.ds(u * 16, 16)]
            for t in range(1, TILES):
                acc = jnp.maximum(acc, stagebuf[t, pl.ds(u * 16, 16)])
            abuf[pl.ds(u * 16, 16)] = acc
        pltpu.sync_copy(abuf, fin_o.at[core, pl.ds(base, NS)])
        plsc.subcore_barrier()
        pltpu.sync_copy(fin_o.at[core], mfin)

        def initz(i, c_):
            for r in range(4):
                banks[r, pl.ds(i * 16, 16)] = _splat(jnp.float32(0.0))
            return c_
        lax.fori_loop(0, N // 16, initz, 0)

        def pex(b, c_):
            for j in range(GP):
                dv = dst_raw[b, pl.ds(j * 16, 16)]
                av = a_ch[pl.ds(b * EB + j * 16, 16)]
                mv = plsc.load_gather(mfin, [dv])
                ev = jnp.exp(av - mv)
                a_ch[pl.ds(b * EB + j * 16, 16)] = ev
                for p in range(4):
                    cur = plsc.load_gather(banks, [blane, dv])
                    plsc.store_scatter(banks, [blane, dv], cur + ev,
                                       mask=bms[p])
            return c_
        lax.fori_loop(0, NB, pex, 0)

        def dred(ci, c_):
            for u in range(NS // 16):
                col = ci * NS + u * 16
                acc = banks[0, pl.ds(col, 16)]
                for r in range(1, 4):
                    acc = acc + banks[r, pl.ds(col, 16)]
                abuf[pl.ds(u * 16, 16)] = acc
            pltpu.sync_copy(abuf, an_o.at[core, sid, pl.ds(ci * NS, NS)])
            return c_
        lax.fori_loop(0, TILES, dred, 0)

        plsc.subcore_barrier()
        pltpu.sync_copy(an_o.at[core, :, pl.ds(base, NS)], stagebuf)
        for u in range(NS // 16):
            acc = stagebuf[0, pl.ds(u * 16, 16)]
            for t in range(1, TILES):
                acc = acc + stagebuf[t, pl.ds(u * 16, 16)]
            abuf[pl.ds(u * 16, 16)] = acc
        pltpu.sync_copy(abuf, fin_o.at[core, pl.ds(base, NS)])
        plsc.subcore_barrier()
        pltpu.sync_copy(fin_o.at[core], mfin)

        def pan(b, c_):
            for j in range(GP):
                dv = dst_raw[b, pl.ds(j * 16, 16)]
                ev = a_ch[pl.ds(b * EB + j * 16, 16)]
                denv = plsc.load_gather(mfin, [dv])
                a_ch[pl.ds(b * EB + j * 16, 16)] = ev / (denv + 1e-16)
            return c_
        lax.fori_loop(0, NB, pan, 0)
        pltpu.sync_copy(a_ch, an_o.at[core, sid])

        for j in range(NS // 16):
            pltpu.sync_copy(zbuf, accum.at[pl.ds(base + j * 16, 16)])
        plsc.subcore_barrier()

        def pw(b, c_):
            cpv = pltpu.async_copy(kv.at[src_stk.at[b]], kvrows, sem1)
            cpv.wait()
            for e in range(EB):
                wv = plsc.load_gather(a_ch, [_splat(b * EB + e)])
                for t in range(dh // 16):
                    kvrows[e, pl.ds(dh + t * 16, 16)] = (
                        kvrows[e, pl.ds(dh + t * 16, 16)] * wv)
            pltpu.sync_copy(kvrows, accum.at[dst_raw.at[b]], add=True)
            return c_
        lax.fori_loop(0, NB, pw, 0)
        plsc.subcore_barrier()
        pltpu.sync_copy(accum.at[pl.ds(base, NS)],
                        attn_o.at[core, pl.ds(base, NS)])

    kern = pl.kernel(
        body,
        out_type=[
            jax.ShapeDtypeStruct((2, TILES, E_T), jnp.float32),
            jax.ShapeDtypeStruct((2, N, RW), jnp.float32),
            jax.ShapeDtypeStruct((2, N), jnp.float32),
        ],
        mesh=mesh,
        compiler_params=_SC_PARAMS,
        scratch_types=[
            pltpu.VMEM((NB, EB), jnp.int32),
            pltpu.VMEM((NB, EB), jnp.int32),
            pltpu.VMEM((NB, EB), jnp.int32),
            pltpu.VMEM((E_T,), jnp.float32),
            pltpu.VMEM((4, N), jnp.float32),
            pltpu.VMEM((32, RW), jnp.float32),
            pltpu.VMEM((32, RW), jnp.float32),
            pltpu.VMEM((32, RW), jnp.float32),
            pltpu.VMEM((32, RW), jnp.float32),
            pltpu.VMEM((EB, RW), jnp.float32),
            pltpu.VMEM((TILES, NS), jnp.float32),
            pltpu.VMEM((NS,), jnp.float32),
            pltpu.VMEM((N,), jnp.float32),
            pltpu.VMEM((16, RW), jnp.float32),
            pltpu.VMEM_SHARED((N, RW), jnp.float32),
            pltpu.SemaphoreType.DMA,
            pltpu.SemaphoreType.DMA,
            pltpu.SemaphoreType.DMA,
            pltpu.SemaphoreType.DMA,
        ],
    )
    return kern


@functools.lru_cache(maxsize=None)
def _make_sc_combine(dh):
    mesh = plsc.VectorSubcoreMesh(core_axis_name="c", subcore_axis_name="s")

    def body(hs, ei, an_i, c_o, diag_o, stg_o,
             dst_stk, src_raw, an_ch, banks, hrows, stagebuf, abuf, zbuf,
             accum, sem1):
        core = lax.axis_index("c")
        sid = lax.axis_index("s")
        lane = jnp.arange(16, dtype=jnp.int32)
        blane = jnp.bitwise_and(lane, 3)
        bms = [jnp.logical_and(lane >= _splat(4 * p), lane < _splat(4 * p + 4))
               for p in range(4)]
        coff = core * N
        base = sid * NS

        pltpu.sync_copy(ei.at[core, 1, sid], dst_stk)
        pltpu.sync_copy(ei.at[core, 0, sid], src_raw)
        pltpu.sync_copy(an_i.at[core, sid], an_ch)

        def p0(b, c_):
            for j in range(GP):
                dv = dst_stk[b, pl.ds(j * 16, 16)]
                dst_stk[b, pl.ds(j * 16, 16)] = dv + _splat(coff)
            return c_
        lax.fori_loop(0, NB, p0, 0)

        def initz(i, c_):
            for r in range(4):
                banks[r, pl.ds(i * 16, 16)] = _splat(jnp.float32(0.0))
            return c_
        lax.fori_loop(0, N // 16, initz, 0)

        for r in range(16):
            for t in range(RW // 16):
                zbuf[r, pl.ds(t * 16, 16)] = jnp.zeros((16,), jnp.float32)
        for j in range(NS // 16):
            pltpu.sync_copy(zbuf, accum.at[pl.ds(base + j * 16, 16)])
        plsc.subcore_barrier()

        def p1(b, c_):
            cph = pltpu.async_copy(hs.at[dst_stk.at[b]], hrows, sem1)
            cph.wait()
            for j in range(GP):
                sv = src_raw[b, pl.ds(j * 16, 16)]
                dvr = dst_stk[b, pl.ds(j * 16, 16)] - _splat(coff)
                av = an_ch[pl.ds(b * EB + j * 16, 16)]
                contrib = jnp.where(sv == dvr, av, jnp.float32(0.0))
                for p in range(4):
                    cur = plsc.load_gather(banks, [blane, sv])
                    plsc.store_scatter(banks, [blane, sv],
                                       cur + contrib, mask=bms[p])
            for e in range(EB):
                wv = plsc.load_gather(an_ch, [_splat(b * EB + e)])
                for t in range(dh // 16):
                    hrows[e, pl.ds(t * 16, 16)] = hrows[e, pl.ds(t * 16, 16)] * wv
            pltpu.sync_copy(hrows, accum.at[src_raw.at[b]], add=True)
            return c_
        lax.fori_loop(0, NB, p1, 0)
        plsc.subcore_barrier()
        pltpu.sync_copy(accum.at[pl.ds(base, NS)],
                        c_o.at[core, pl.ds(base, NS)])

        def dred(ci, c_):
            for u in range(NS // 16):
                col = ci * NS + u * 16
                acc = banks[0, pl.ds(col, 16)]
                for r in range(1, 4):
                    acc = acc + banks[r, pl.ds(col, 16)]
                abuf[pl.ds(u * 16, 16)] = acc
            pltpu.sync_copy(abuf, stg_o.at[core, sid, pl.ds(ci * NS, NS)])
            return c_
        lax.fori_loop(0, TILES, dred, 0)

        plsc.subcore_barrier()
        pltpu.sync_copy(stg_o.at[core, :, pl.ds(base, NS)], stagebuf)
        for u in range(NS // 16):
            acc = stagebuf[0, pl.ds(u * 16, 16)]
            for t in range(1, TILES):
                acc = acc + stagebuf[t, pl.ds(u * 16, 16)]
            abuf[pl.ds(u * 16, 16)] = acc
        pltpu.sync_copy(abuf, diag_o.at[core, pl.ds(base, NS)])

    kern = pl.kernel(
        body,
        out_type=[
            jax.ShapeDtypeStruct((2, N, RW), jnp.float32),
            jax.ShapeDtypeStruct((2, N), jnp.float32),
            jax.ShapeDtypeStruct((2, TILES, N), jnp.float32),
        ],
        mesh=mesh,
        compiler_params=_SC_PARAMS,
        scratch_types=[
            pltpu.VMEM((NB, EB), jnp.int32),
            pltpu.VMEM((NB, EB), jnp.int32),
            pltpu.VMEM((E_T,), jnp.float32),
            pltpu.VMEM((4, N), jnp.float32),
            pltpu.VMEM((EB, RW), jnp.float32),
            pltpu.VMEM((TILES, NS), jnp.float32),
            pltpu.VMEM((NS,), jnp.float32),
            pltpu.VMEM((16, RW), jnp.float32),
            pltpu.VMEM_SHARED((N, RW), jnp.float32),
            pltpu.SemaphoreType.DMA,
        ],
    )
    return kern


def _pack_ei(a, b):
    return jnp.stack([a, b], axis=0).reshape(2, 2, TILES, NB, EB)


def _stack2(a, b=None):
    if b is None:
        b = a
    return jnp.concatenate([a, b], axis=0)


def kernel(x, enhanced_weights, enhanced_index, adj, adj_prue, training,
           enc1_W, enc1_b, enc1_gamma, enc1_beta, enc2_W, enc2_b, enc2_gamma, enc2_beta,
           gc1_Wq, gc1_bq, gc1_Wk, gc1_bk, gc1_Wv, gc1_bv, gc1_Ws, gc1_bs,
           ch0_Wq, ch0_bq, ch0_Wk, ch0_bk, ch0_Wv, ch0_bv, ch0_Ws, ch0_bs,
           gc2_Wq, gc2_bq, gc2_Wk, gc2_bk, gc2_Wv, gc2_bv, gc2_Ws, gc2_bs,
           dec_W, dec_b, dec_gamma, dec_beta, cluster):
    adj = adj.astype(jnp.int32)
    adj_prue = adj_prue.astype(jnp.int32)
    ei_pair = _pack_ei(adj, adj_prue)
    ei_a = _pack_ei(adj, adj)
    ei_p = _pack_ei(adj_prue, adj_prue)

    feat_x = _tc_encoder(x, enc1_W, enc1_b, enc1_gamma, enc1_beta,
                         enc2_W, enc2_b, enc2_gamma, enc2_beta)

    q, kv, s = _tc_proj(feat_x, gc1_Wq, gc1_bq, gc1_Wk, gc1_bk,
                        gc1_Wv, gc1_bv, gc1_Ws, gc1_bs)
    an1, attn1, _ = _make_sc_attn(64)(_stack2(q), _stack2(kv), ei_pair)
    h1 = _tc_skip(attn1[0], s, 64)
    h1p = _tc_skip(attn1[1], s, 64)
    c1, d1, _ = _make_sc_combine(64)(_stack2(h1, h1p), ei_pair, an1)
    xh = _tc_merge(c1, d1, h1, h1p, 64, relu=True)

    qa, kva, sa = _tc_proj(xh, ch0_Wq, ch0_bq, ch0_Wk, ch0_bk,
                           ch0_Wv, ch0_bv, ch0_Ws, ch0_bs)
    anA, attnA, _ = _make_sc_attn(64)(_stack2(qa), _stack2(kva), ei_a)
    x1 = _tc_skip(attnA[0], sa, 64)
    qb, kvb, sb = _tc_proj(x1[:, 0:64], ch0_Wq, ch0_bq, ch0_Wk, ch0_bk,
                           ch0_Wv, ch0_bv, ch0_Ws, ch0_bs)
    anB, attnB, _ = _make_sc_attn(64)(_stack2(qb), _stack2(kvb), ei_p)
    xp = _tc_skip(attnB[0], sb, 64)
    an2 = jnp.stack([anA[0], anB[0]], axis=0)
    c2, d2, _ = _make_sc_combine(64)(_stack2(x1, xp), ei_pair, an2)
    xh = _tc_merge(c2, d2, x1, xp, 64, relu=True)

    q3, kv3, s3 = _tc_proj(xh, gc2_Wq, gc2_bq, gc2_Wk, gc2_bk,
                           gc2_Wv, gc2_bv, gc2_Ws, gc2_bs)
    an3, attn3, _ = _make_sc_attn(32)(_stack2(q3), _stack2(kv3), ei_pair)
    mu1 = _tc_skip(attn3[0], s3, 32)
    mup = _tc_skip(attn3[1], s3, 32)
    c3, d3, _ = _make_sc_combine(32)(_stack2(mu1, mup), ei_pair, an3)
    mu = _tc_merge(c3, d3, mu1, mup, 32, relu=False)

    z, de_feat, qc = _tc_final(feat_x, mu, dec_W, dec_b, dec_gamma, dec_beta,
                               cluster)
    return (z, de_feat, qc, feat_x, z)


# speedup vs baseline: 7.1789x; 1.0184x over previous
import functools

import jax
import jax.numpy as jnp
from jax import lax
from jax.experimental import pallas as pl
from jax.experimental.pallas import tpu as pltpu
from jax.experimental.pallas import tpu_sc as plsc

N = 4096
E = 65536
AT = 0.5
TILES = 16
E_T = E // TILES
EB = 128
NB = E_T // EB
GP = EB // 16
NS = N // TILES
RW = 128
AW = 32

_SC_PARAMS = pltpu.CompilerParams(needs_layout_passes=False)


def _matmul_t(x, w):
    return lax.dot_general(x, w, (((1,), (1,)), ((), ())),
                           preferred_element_type=jnp.float32)


def _norm_elu(h, gamma, beta):
    m = jnp.mean(h, axis=0)
    v = jnp.mean((h - m) ** 2, axis=0)
    h = (h - m) / jnp.sqrt(v + 1e-4) * gamma + beta
    return jnp.where(h > 0, h, jnp.exp(h) - 1.0)


def _tc_encoder(x, w1, b1, g1, be1, w2, b2, g2, be2):
    def body(x_r, w1_r, b1_r, g1_r, be1_r, w2_r, b2_r, g2_r, be2_r, o_r):
        h = _matmul_t(x_r[...], w1_r[...]) + b1_r[...]
        h = _norm_elu(h, g1_r[...], be1_r[...])
        h = _matmul_t(h, w2_r[...]) + b2_r[...]
        o_r[...] = _norm_elu(h, g2_r[...], be2_r[...])

    return pl.pallas_call(
        body,
        out_shape=jax.ShapeDtypeStruct((N, w2.shape[0]), jnp.float32),
    )(x, w1, b1, g1, be1, w2, b2, g2, be2)


def _tc_proj(x, wq, bq, wk, bk, wv, bv, ws, bs):
    dh = wq.shape[0]
    scale = 1.0 / float(dh) ** 0.5

    def body(x_r, wq_r, bq_r, wk_r, bk_r, wv_r, bv_r, ws_r, bs_r,
             q_r, kv_r, s_r):
        xv = x_r[...]
        q = (_matmul_t(xv, wq_r[...]) + bq_r[...]) * scale
        k = _matmul_t(xv, wk_r[...]) + bk_r[...]
        v = _matmul_t(xv, wv_r[...]) + bv_r[...]
        q_r[...] = jnp.concatenate(
            [q, jnp.zeros((N, RW - dh), jnp.float32)], axis=1)
        if 2 * dh == RW:
            kv_r[...] = jnp.concatenate([k, v], axis=1)
        else:
            kv_r[...] = jnp.concatenate(
                [k, v, jnp.zeros((N, RW - 2 * dh), jnp.float32)], axis=1)
        s_r[...] = _matmul_t(xv, ws_r[...]) + bs_r[...]

    return pl.pallas_call(body, out_shape=[
        jax.ShapeDtypeStruct((N, RW), jnp.float32),
        jax.ShapeDtypeStruct((N, RW), jnp.float32),
        jax.ShapeDtypeStruct((N, dh), jnp.float32),
    ])(x, wq, bq, wk, bk, wv, bv, ws, bs)


def _tc_skip(attn, s, dh):
    def body(a_r, s_r, o_r):
        h = a_r[:, dh:2 * dh] + s_r[...]
        o_r[...] = jnp.concatenate(
            [h, jnp.zeros((N, RW - dh), jnp.float32)], axis=1)

    return pl.pallas_call(
        body, out_shape=jax.ShapeDtypeStruct((N, RW), jnp.float32))(attn, s)


def _tc_merge(c, diag, ha, hp, dh, relu):
    def body(c_r, d_r, ha_r, hp_r, o_r):
        d0 = d_r[0, :].reshape(N, 1)
        d1 = d_r[1, :].reshape(N, 1)
        out = ((1.0 - AT) * (c_r[0, :, 0:dh] - d0 * ha_r[:, 0:dh])
               + AT * (c_r[1, :, 0:dh] - d1 * hp_r[:, 0:dh]))
        if relu:
            out = jnp.maximum(out, 0.0)
        o_r[...] = out

    return pl.pallas_call(
        body, out_shape=jax.ShapeDtypeStruct((N, dh), jnp.float32))(
            c, diag, ha, hp)


def _tc_final(feat_x, mu, dw, db, dg, dbe, cluster):
    nclust = cluster.shape[0]

    def body(f_r, mu_r, dw_r, db_r, dg_r, dbe_r, cl_r, z_r, de_r, q_r):
        z = jnp.concatenate([f_r[...], mu_r[...]], axis=1)
        z_r[...] = z
        h = _matmul_t(z, dw_r[...]) + db_r[...]
        de_r[...] = _norm_elu(h, dg_r[...], dbe_r[...])
        cl = cl_r[...]
        cols = []
        for j in range(nclust):
            dif = z - cl[j, :].reshape(1, -1)
            cols.append(1.0 / (1.0 + jnp.sum(dif * dif, axis=1, keepdims=True)))
        q = jnp.concatenate(cols, axis=1)
        q_r[...] = q / jnp.sum(q, axis=1, keepdims=True)

    d = feat_x.shape[1] + mu.shape[1]
    return pl.pallas_call(
        body,
        out_shape=[
            jax.ShapeDtypeStruct((N, d), jnp.float32),
            jax.ShapeDtypeStruct((N, dw.shape[0]), jnp.float32),
            jax.ShapeDtypeStruct((N, nclust), jnp.float32),
        ],
    )(feat_x, mu, dw, db, dg, dbe, cluster)


def _splat(val):
    return jnp.full((16,), val)


@functools.lru_cache(maxsize=None)
def _make_sc_attn(dh):
    mesh = plsc.VectorSubcoreMesh(core_axis_name="c", subcore_axis_name="s")

    def body(qs, kv, ei, an_o, attn_o, fin_o,
             dst_stk, dst_raw, src_stk, a_ch, banks,
             qrows, qrows2, kvrows2a, kvrows2b, kvrows, kvrowsb, stagebuf,
             abuf, mfin, zbuf, accum, sem1, sem2, sem3, sem4):
        core = lax.axis_index("c")
        sid = lax.axis_index("s")
        lane = jnp.arange(16, dtype=jnp.int32)
        blane = jnp.bitwise_and(lane, 1)
        bms = [jnp.logical_and(lane >= _splat(2 * p), lane < _splat(2 * p + 2))
               for p in range(8)]
        coff = core * N
        base = sid * NS

        pltpu.sync_copy(ei.at[core, 1, sid], dst_raw)
        pltpu.sync_copy(ei.at[core, 0, sid], src_stk)

        def p0(b, c_):
            for j in range(GP):
                dv = dst_raw[b, pl.ds(j * 16, 16)]
                dst_stk[b, pl.ds(j * 16, 16)] = dv + _splat(coff)
                sv = src_stk[b, pl.ds(j * 16, 16)]
                src_stk[b, pl.ds(j * 16, 16)] = sv + _splat(coff)
            return c_
        lax.fori_loop(0, NB, p0, 0)

        for r in range(16):
            for t in range(RW // 16):
                zbuf[r, pl.ds(t * 16, 16)] = jnp.zeros((16,), jnp.float32)

        def p1(b, c_):
            qb = (qrows, qrows2)
            kb = (kvrows2a, kvrows2b)
            sq = (sem1, sem3)
            sk = (sem2, sem4)
            cps = [None, None]
            cps[0] = (pltpu.async_copy(
                          qs.at[dst_stk.at[b, pl.ds(0, 32)]], qb[0], sq[0]),
                      pltpu.async_copy(
                          kv.at[src_stk.at[b, pl.ds(0, 32)]], kb[0], sk[0]))
            for o in range(4):
                cur = o % 2
                if o < 3:
                    nxt = 1 - cur
                    cps[nxt] = (
                        pltpu.async_copy(
                            qs.at[dst_stk.at[b, pl.ds((o + 1) * 32, 32)]],
                            qb[nxt], sq[nxt]),
                        pltpu.async_copy(
                            kv.at[src_stk.at[b, pl.ds((o + 1) * 32, 32)]],
                            kb[nxt], sk[nxt]))
                cps[cur][0].wait()
                cps[cur][1].wait()
                for g in range(2):
                    ev = lane + _splat(g * 16)
                    acc = (plsc.load_gather(qb[cur], [ev, _splat(0)])
                           * plsc.load_gather(kb[cur], [ev, _splat(0)]))
                    for d in range(1, dh):
                        acc = acc + (plsc.load_gather(qb[cur], [ev, _splat(d)])
                                     * plsc.load_gather(kb[cur], [ev, _splat(d)]))
                    a_ch[pl.ds(b * EB + o * 32 + g * 16, 16)] = acc
            return c_
        lax.fori_loop(0, NB, p1, 0)

        def initm(i, c_):
            for r in range(2):
                banks[r, pl.ds(i * 16, 16)] = _splat(jnp.float32(-jnp.inf))
            return c_
        lax.fori_loop(0, N // 16, initm, 0)

        def smax(b, c_):
            for j in range(GP):
                dv = dst_raw[b, pl.ds(j * 16, 16)]
                av = a_ch[pl.ds(b * EB + j * 16, 16)]
                for p in range(8):
                    cur = plsc.load_gather(banks, [blane, dv])
                    plsc.store_scatter(banks, [blane, dv],
                                       jnp.maximum(cur, av), mask=bms[p])
            return c_
        lax.fori_loop(0, NB, smax, 0)

        def mred(ci, c_):
            for u in range(NS // 16):
                col = ci * NS + u * 16
                acc = banks[0, pl.ds(col, 16)]
                acc = jnp.maximum(acc, banks[1, pl.ds(col, 16)])
                abuf[pl.ds(u * 16, 16)] = acc
            pltpu.sync_copy(abuf, an_o.at[core, sid, pl.ds(ci * NS, NS)])
            return c_
        lax.fori_loop(0, TILES, mred, 0)

        plsc.subcore_barrier()
        pltpu.sync_copy(an_o.at[core, :, pl.ds(base, NS)], stagebuf)
        for u in range(NS // 16):
            acc = stagebuf[0, pl.ds(u * 16, 16)]
            for t in range(1, TILES):
                acc = jnp.maximum(acc, stagebuf[t, pl.ds(u * 16, 16)])
            abuf[pl.ds(u * 16, 16)] = acc
        pltpu.sync_copy(abuf, fin_o.at[core, pl.ds(base, NS)])
        plsc.subcore_barrier()
        pltpu.sync_copy(fin_o.at[core], mfin)

        def initz(i, c_):
            for r in range(2):
                banks[r, pl.ds(i * 16, 16)] = _splat(jnp.float32(0.0))
            return c_
        lax.fori_loop(0, N // 16, initz, 0)

        def pex(b, c_):
            for j in range(GP):
                dv = dst_raw[b, pl.ds(j * 16, 16)]
                av = a_ch[pl.ds(b * EB + j * 16, 16)]
                mv = plsc.load_gather(mfin, [dv])
                ev = jnp.exp(av - mv)
                a_ch[pl.ds(b * EB + j * 16, 16)] = ev
                for p in range(8):
                    cur = plsc.load_gather(banks, [blane, dv])
                    plsc.store_scatter(banks, [blane, dv], cur + ev,
                                       mask=bms[p])
            return c_
        lax.fori_loop(0, NB, pex, 0)

        def dred(ci, c_):
            for u in range(NS // 16):
                col = ci * NS + u * 16
                acc = banks[0, pl.ds(col, 16)]
                acc = acc + banks[1, pl.ds(col, 16)]
                abuf[pl.ds(u * 16, 16)] = acc
            pltpu.sync_copy(abuf, an_o.at[core, sid, pl.ds(ci * NS, NS)])
            return c_
        lax.fori_loop(0, TILES, dred, 0)

        plsc.subcore_barrier()
        pltpu.sync_copy(an_o.at[core, :, pl.ds(base, NS)], stagebuf)
        for u in range(NS // 16):
            acc = stagebuf[0, pl.ds(u * 16, 16)]
            for t in range(1, TILES):
                acc = acc + stagebuf[t, pl.ds(u * 16, 16)]
            abuf[pl.ds(u * 16, 16)] = acc
        pltpu.sync_copy(abuf, fin_o.at[core, pl.ds(base, NS)])
        plsc.subcore_barrier()
        pltpu.sync_copy(fin_o.at[core], mfin)

        def pan(b, c_):
            for j in range(GP):
                dv = dst_raw[b, pl.ds(j * 16, 16)]
                ev = a_ch[pl.ds(b * EB + j * 16, 16)]
                denv = plsc.load_gather(mfin, [dv])
                a_ch[pl.ds(b * EB + j * 16, 16)] = ev / (denv + 1e-16)
            return c_
        lax.fori_loop(0, NB, pan, 0)
        pltpu.sync_copy(a_ch, an_o.at[core, sid])

        for j in range(NS // 16):
            pltpu.sync_copy(zbuf, accum.at[pl.ds(base + j * 16, 16)])
        plsc.subcore_barrier()

        pltpu.async_copy(kv.at[src_stk.at[0]], kvrows, sem1)

        def pw(i, c_):
            bufs = (kvrows, kvrowsb)
            sems = (sem1, sem2)
            for ph in range(2):
                b = i * 2 + ph
                nb = jnp.minimum(b + 1, NB - 1)
                pltpu.async_copy(kv.at[src_stk.at[nb]], bufs[1 - ph],
                                 sems[1 - ph])
                pltpu.make_async_copy(kv.at[src_stk.at[b]], bufs[ph],
                                      sems[ph]).wait()
                for e in range(EB):
                    wv = plsc.load_gather(a_ch, [_splat(b * EB + e)])
                    for t in range(dh // 16):
                        bufs[ph][e, pl.ds(dh + t * 16, 16)] = (
                            bufs[ph][e, pl.ds(dh + t * 16, 16)] * wv)
                pltpu.sync_copy(bufs[ph], accum.at[dst_raw.at[b]], add=True)
            return c_
        lax.fori_loop(0, NB // 2, pw, 0)
        pltpu.make_async_copy(kv.at[src_stk.at[0]], kvrows,
                              sem1).wait()
        plsc.subcore_barrier()
        pltpu.sync_copy(accum.at[pl.ds(base, NS)],
                        attn_o.at[core, pl.ds(base, NS)])

    kern = pl.kernel(
        body,
        out_type=[
            jax.ShapeDtypeStruct((2, TILES, E_T), jnp.float32),
            jax.ShapeDtypeStruct((2, N, RW), jnp.float32),
            jax.ShapeDtypeStruct((2, N), jnp.float32),
        ],
        mesh=mesh,
        compiler_params=_SC_PARAMS,
        scratch_types=[
            pltpu.VMEM((NB, EB), jnp.int32),
            pltpu.VMEM((NB, EB), jnp.int32),
            pltpu.VMEM((NB, EB), jnp.int32),
            pltpu.VMEM((E_T,), jnp.float32),
            pltpu.VMEM((2, N), jnp.float32),
            pltpu.VMEM((32, RW), jnp.float32),
            pltpu.VMEM((32, RW), jnp.float32),
            pltpu.VMEM((32, RW), jnp.float32),
            pltpu.VMEM((32, RW), jnp.float32),
            pltpu.VMEM((EB, RW), jnp.float32),
            pltpu.VMEM((EB, RW), jnp.float32),
            pltpu.VMEM((TILES, NS), jnp.float32),
            pltpu.VMEM((NS,), jnp.float32),
            pltpu.VMEM((N,), jnp.float32),
            pltpu.VMEM((16, RW), jnp.float32),
            pltpu.VMEM_SHARED((N, RW), jnp.float32),
            pltpu.SemaphoreType.DMA,
            pltpu.SemaphoreType.DMA,
            pltpu.SemaphoreType.DMA,
            pltpu.SemaphoreType.DMA,
        ],
    )
    return kern


@functools.lru_cache(maxsize=None)
def _make_sc_combine(dh):
    mesh = plsc.VectorSubcoreMesh(core_axis_name="c", subcore_axis_name="s")

    def body(hs, ei, an_i, c_o, diag_o, stg_o,
             dst_stk, src_raw, an_ch, banks, hrows, hrowsb, stagebuf, abuf,
             zbuf, accum, sem1, sem2):
        core = lax.axis_index("c")
        sid = lax.axis_index("s")
        lane = jnp.arange(16, dtype=jnp.int32)
        blane = jnp.bitwise_and(lane, 1)
        bms = [jnp.logical_and(lane >= _splat(2 * p), lane < _splat(2 * p + 2))
               for p in range(8)]
        coff = core * N
        base = sid * NS

        pltpu.sync_copy(ei.at[core, 1, sid], dst_stk)
        pltpu.sync_copy(ei.at[core, 0, sid], src_raw)
        pltpu.sync_copy(an_i.at[core, sid], an_ch)

        def p0(b, c_):
            for j in range(GP):
                dv = dst_stk[b, pl.ds(j * 16, 16)]
                dst_stk[b, pl.ds(j * 16, 16)] = dv + _splat(coff)
            return c_
        lax.fori_loop(0, NB, p0, 0)

        def initz(i, c_):
            for r in range(2):
                banks[r, pl.ds(i * 16, 16)] = _splat(jnp.float32(0.0))
            return c_
        lax.fori_loop(0, N // 16, initz, 0)

        for r in range(16):
            for t in range(RW // 16):
                zbuf[r, pl.ds(t * 16, 16)] = jnp.zeros((16,), jnp.float32)
        for j in range(NS // 16):
            pltpu.sync_copy(zbuf, accum.at[pl.ds(base + j * 16, 16)])
        plsc.subcore_barrier()

        pltpu.async_copy(hs.at[dst_stk.at[0]], hrows, sem1)

        def p1(i, c_):
            bufs = (hrows, hrowsb)
            sems = (sem1, sem2)
            for ph in range(2):
                b = i * 2 + ph
                nb = jnp.minimum(b + 1, NB - 1)
                pltpu.async_copy(hs.at[dst_stk.at[nb]], bufs[1 - ph],
                                 sems[1 - ph])
                pltpu.make_async_copy(hs.at[dst_stk.at[b]], bufs[ph],
                                      sems[ph]).wait()
                for j in range(GP):
                    sv = src_raw[b, pl.ds(j * 16, 16)]
                    dvr = dst_stk[b, pl.ds(j * 16, 16)] - _splat(coff)
                    av = an_ch[pl.ds(b * EB + j * 16, 16)]
                    contrib = jnp.where(sv == dvr, av, jnp.float32(0.0))
                    for p in range(8):
                        cur = plsc.load_gather(banks, [blane, sv])
                        plsc.store_scatter(banks, [blane, sv],
                                           cur + contrib, mask=bms[p])
                for e in range(EB):
                    wv = plsc.load_gather(an_ch, [_splat(b * EB + e)])
                    for t in range(dh // 16):
                        bufs[ph][e, pl.ds(t * 16, 16)] = (
                            bufs[ph][e, pl.ds(t * 16, 16)] * wv)
                pltpu.sync_copy(bufs[ph], accum.at[src_raw.at[b]], add=True)
            return c_
        lax.fori_loop(0, NB // 2, p1, 0)
        pltpu.make_async_copy(hs.at[dst_stk.at[0]], hrows, sem1).wait()
        plsc.subcore_barrier()
        pltpu.sync_copy(accum.at[pl.ds(base, NS)],
                        c_o.at[core, pl.ds(base, NS)])

        def dred(ci, c_):
            for u in range(NS // 16):
                col = ci * NS + u * 16
                acc = banks[0, pl.ds(col, 16)]
                acc = acc + banks[1, pl.ds(col, 16)]
                abuf[pl.ds(u * 16, 16)] = acc
            pltpu.sync_copy(abuf, stg_o.at[core, sid, pl.ds(ci * NS, NS)])
            return c_
        lax.fori_loop(0, TILES, dred, 0)

        plsc.subcore_barrier()
        pltpu.sync_copy(stg_o.at[core, :, pl.ds(base, NS)], stagebuf)
        for u in range(NS // 16):
            acc = stagebuf[0, pl.ds(u * 16, 16)]
            for t in range(1, TILES):
                acc = acc + stagebuf[t, pl.ds(u * 16, 16)]
            abuf[pl.ds(u * 16, 16)] = acc
        pltpu.sync_copy(abuf, diag_o.at[core, pl.ds(base, NS)])

    kern = pl.kernel(
        body,
        out_type=[
            jax.ShapeDtypeStruct((2, N, RW), jnp.float32),
            jax.ShapeDtypeStruct((2, N), jnp.float32),
            jax.ShapeDtypeStruct((2, TILES, N), jnp.float32),
        ],
        mesh=mesh,
        compiler_params=_SC_PARAMS,
        scratch_types=[
            pltpu.VMEM((NB, EB), jnp.int32),
            pltpu.VMEM((NB, EB), jnp.int32),
            pltpu.VMEM((E_T,), jnp.float32),
            pltpu.VMEM((2, N), jnp.float32),
            pltpu.VMEM((EB, RW), jnp.float32),
            pltpu.VMEM((EB, RW), jnp.float32),
            pltpu.VMEM((TILES, NS), jnp.float32),
            pltpu.VMEM((NS,), jnp.float32),
            pltpu.VMEM((16, RW), jnp.float32),
            pltpu.VMEM_SHARED((N, RW), jnp.float32),
            pltpu.SemaphoreType.DMA,
            pltpu.SemaphoreType.DMA,
        ],
    )
    return kern


def _pack_ei(a, b):
    return jnp.stack([a, b], axis=0).reshape(2, 2, TILES, NB, EB)


def _stack2(a, b=None):
    if b is None:
        b = a
    return jnp.concatenate([a, b], axis=0)


def kernel(x, enhanced_weights, enhanced_index, adj, adj_prue, training,
           enc1_W, enc1_b, enc1_gamma, enc1_beta, enc2_W, enc2_b, enc2_gamma, enc2_beta,
           gc1_Wq, gc1_bq, gc1_Wk, gc1_bk, gc1_Wv, gc1_bv, gc1_Ws, gc1_bs,
           ch0_Wq, ch0_bq, ch0_Wk, ch0_bk, ch0_Wv, ch0_bv, ch0_Ws, ch0_bs,
           gc2_Wq, gc2_bq, gc2_Wk, gc2_bk, gc2_Wv, gc2_bv, gc2_Ws, gc2_bs,
           dec_W, dec_b, dec_gamma, dec_beta, cluster):
    adj = adj.astype(jnp.int32)
    adj_prue = adj_prue.astype(jnp.int32)
    ei_pair = _pack_ei(adj, adj_prue)
    ei_a = _pack_ei(adj, adj)
    ei_p = _pack_ei(adj_prue, adj_prue)

    feat_x = _tc_encoder(x, enc1_W, enc1_b, enc1_gamma, enc1_beta,
                         enc2_W, enc2_b, enc2_gamma, enc2_beta)

    q, kv, s = _tc_proj(feat_x, gc1_Wq, gc1_bq, gc1_Wk, gc1_bk,
                        gc1_Wv, gc1_bv, gc1_Ws, gc1_bs)
    an1, attn1, _ = _make_sc_attn(64)(_stack2(q), _stack2(kv), ei_pair)
    h1 = _tc_skip(attn1[0], s, 64)
    h1p = _tc_skip(attn1[1], s, 64)
    c1, d1, _ = _make_sc_combine(64)(_stack2(h1, h1p), ei_pair, an1)
    xh = _tc_merge(c1, d1, h1, h1p, 64, relu=True)

    qa, kva, sa = _tc_proj(xh, ch0_Wq, ch0_bq, ch0_Wk, ch0_bk,
                           ch0_Wv, ch0_bv, ch0_Ws, ch0_bs)
    anA, attnA, _ = _make_sc_attn(64)(_stack2(qa), _stack2(kva), ei_a)
    x1 = _tc_skip(attnA[0], sa, 64)
    qb, kvb, sb = _tc_proj(x1[:, 0:64], ch0_Wq, ch0_bq, ch0_Wk, ch0_bk,
                           ch0_Wv, ch0_bv, ch0_Ws, ch0_bs)
    anB, attnB, _ = _make_sc_attn(64)(_stack2(qb), _stack2(kvb), ei_p)
    xp = _tc_skip(attnB[0], sb, 64)
    an2 = jnp.stack([anA[0], anB[0]], axis=0)
    c2, d2, _ = _make_sc_combine(64)(_stack2(x1, xp), ei_pair, an2)
    xh = _tc_merge(c2, d2, x1, xp, 64, relu=True)

    q3, kv3, s3 = _tc_proj(xh, gc2_Wq, gc2_bq, gc2_Wk, gc2_bk,
                           gc2_Wv, gc2_bv, gc2_Ws, gc2_bs)
    an3, attn3, _ = _make_sc_attn(32)(_stack2(q3), _stack2(kv3), ei_pair)
    mu1 = _tc_skip(attn3[0], s3, 32)
    mup = _tc_skip(attn3[1], s3, 32)
    c3, d3, _ = _make_sc_combine(32)(_stack2(mu1, mup), ei_pair, an3)
    mu = _tc_merge(c3, d3, mu1, mup, 32, relu=False)

    z, de_feat, qc = _tc_final(feat_x, mu, dec_W, dec_b, dec_gamma, dec_beta,
                               cluster)
    return (z, de_feat, qc, feat_x, z)
